# baseline (speedup 1.0000x reference)
import functools

import jax
import jax.numpy as jnp
from jax import lax
from jax.experimental import pallas as pl
from jax.experimental.pallas import tpu as pltpu
from jax.experimental.pallas import tpu_sc as plsc

B = 128
NUM_COLS = 256
NUM_NODES = 128
NUM_CATS = 100
CATS_PAD = 128
EMB = 64
HID = 64
PERIODS = 37
NUM_EDGES = 4096

_F32 = jnp.float32
_WIDE = 2 * HID
_CBLK = 8
_NSTEPS = NUM_COLS // _CBLK


def _tc_prep(emb3, x_batch, Wz, LzWt, bz, Lzb, Wh, LhWt, bh, Lhb,
             dst_row, src_col, dst_col):
    def body(emb_ref, x_ref, wz_ref, lzt_ref, bz_ref, lzb_ref, wh_ref,
             lht_ref, bh_ref, lhb_ref, dstr_ref, srcc_ref, dstc_ref,
             tzh_ref, amat_ref, czh_ref, xst_ref):
        mz = jnp.dot(wz_ref[...], lzt_ref[...], preferred_element_type=_F32)
        mh = jnp.dot(wh_ref[...], lht_ref[...], preferred_element_type=_F32)
        mzh = jnp.concatenate([mz, mh], axis=1)
        pad = jnp.zeros((CATS_PAD - NUM_CATS, _WIDE), _F32)
        for i in range(_CBLK):
            proj = jnp.dot(emb_ref[i], mzh, preferred_element_type=_F32)
            tzh_ref[i] = jnp.concatenate([proj, pad], axis=0)

        @pl.when(pl.program_id(0) == 0)
        def _():
            cz = jnp.dot(bz_ref[...], lzt_ref[...],
                         preferred_element_type=_F32) + lzb_ref[...]
            ch = jnp.dot(bh_ref[...], lht_ref[...],
                         preferred_element_type=_F32) + lhb_ref[...]
            czh_ref[...] = jnp.concatenate([cz, ch], axis=1)

            dstr = dstr_ref[...].astype(jnp.int32)
            srcc = srcc_ref[...].astype(jnp.int32)
            dstc = dstc_ref[...].astype(jnp.int32)
            iota_i = lax.broadcasted_iota(jnp.int32, (NUM_NODES, NUM_EDGES), 0)
            odt = (iota_i == dstr).astype(_F32)
            iota_j = lax.broadcasted_iota(jnp.int32, (NUM_EDGES, NUM_NODES), 1)
            os_ = (iota_j == srcc).astype(_F32)
            od2 = (iota_j == dstc).astype(_F32)
            cnt = jnp.dot(odt, os_, preferred_element_type=_F32)
            deg_col = jnp.sum(odt, axis=1, keepdims=True) + 1.0
            deg_row = jnp.sum(od2, axis=0, keepdims=True) + 1.0
            dinv_col = lax.rsqrt(deg_col)
            dinv_row = lax.rsqrt(deg_row)
            ii = lax.broadcasted_iota(jnp.int32, (NUM_NODES, NUM_NODES), 0)
            jj = lax.broadcasted_iota(jnp.int32, (NUM_NODES, NUM_NODES), 1)
            eye = (ii == jj).astype(_F32)
            amat_ref[...] = (cnt + eye) * dinv_col * dinv_row

            xt = jnp.transpose(x_ref[...].astype(_F32)).astype(jnp.int32)
            xst_ref[...] = xt + CATS_PAD * lax.broadcasted_iota(
                jnp.int32, (NUM_COLS, B), 0)

    grid = (_NSTEPS,)
    full = lambda shape: pl.BlockSpec(shape, lambda g: tuple(0 for _ in shape))
    return pl.pallas_call(
        body,
        grid=grid,
        in_specs=[
            pl.BlockSpec((_CBLK, NUM_CATS, EMB), lambda g: (g, 0, 0)),
            full((B, NUM_COLS)),
            full((EMB, HID)), full((HID, HID)), full((1, HID)), full((1, HID)),
            full((EMB, HID)), full((HID, HID)), full((1, HID)), full((1, HID)),
            full((1, NUM_EDGES)), full((NUM_EDGES, 1)), full((NUM_EDGES, 1)),
        ],
        out_specs=[
            pl.BlockSpec((_CBLK, CATS_PAD, _WIDE), lambda g: (g, 0, 0)),
            full((NUM_NODES, NUM_NODES)),
            full((1, _WIDE)),
            full((NUM_COLS, B)),
        ],
        out_shape=(
            jax.ShapeDtypeStruct((NUM_COLS, CATS_PAD, _WIDE), _F32),
            jax.ShapeDtypeStruct((NUM_NODES, NUM_NODES), _F32),
            jax.ShapeDtypeStruct((1, _WIDE), _F32),
            jax.ShapeDtypeStruct((NUM_COLS, B), jnp.int32),
        ),
    )(emb3, x_batch, Wz, LzWt, bz, Lzb, Wh, LhWt, bh, Lhb,
      dst_row, src_col, dst_col)


def _sc_gather(cols_ad_dis, xs_t, tzh):
    info = plsc.get_sparse_core_info()
    nc = info.num_cores
    rows_total = 2 * NUM_NODES
    npw = 8
    half = npw // 2
    nworkers = rows_total // npw

    mesh = plsc.VectorSubcoreMesh(core_axis_name="c", subcore_axis_name="s")

    @functools.partial(
        pl.kernel,
        mesh=mesh,
        compiler_params=pltpu.CompilerParams(use_tc_tiling_on_sc=False),
        out_type=jax.ShapeDtypeStruct((rows_total * B, _WIDE), _F32),
        scratch_types=[
            pltpu.VMEM((npw,), jnp.int32),
            pltpu.VMEM((npw, B), jnp.int32),
            pltpu.VMEM((half * B, _WIDE), _F32),
            pltpu.SemaphoreType.DMA,
        ],
    )
    def body(cols_h, xst_h, tzh_h, out_h, myc_v, idx_v, rows_v, sem):
        wid = lax.axis_index("s") * nc + lax.axis_index("c")

        @pl.when(wid < nworkers)
        def _():
            base = wid * npw
            pltpu.sync_copy(cols_h.at[pl.ds(base, npw)], myc_v)
            pltpu.async_copy(xst_h.at[myc_v], idx_v, sem).wait()
            for p in range(2):
                cps = [
                    pltpu.async_copy(
                        tzh_h.at[idx_v.at[p * half + j]],
                        rows_v.at[pl.ds(j * B, B)],
                        sem,
                    )
                    for j in range(half)
                ]
                for cp in cps:
                    cp.wait()
                pltpu.sync_copy(
                    rows_v,
                    out_h.at[pl.ds((base + p * half) * B, half * B)],
                )

    return body(cols_ad_dis, xs_t, tzh)


def _tc_main(g_wide, amat, czh, att, los, C1W, C1b, C2W, C2b):
    lanes = B * _WIDE

    def body(g_ref, amat_ref, czh_ref, att_ref, los_ref,
             c1w_ref, c1b_ref, c2w_ref, c2b_ref, out_ref):
        amat = amat_ref[...]
        lane_mod = lax.rem(lax.broadcasted_iota(jnp.int32, (_WIDE, lanes), 1),
                           _WIDE)
        sel_bias = (lane_mod == lax.broadcasted_iota(
            jnp.int32, (_WIDE, lanes), 0)).astype(_F32)
        czh_wide = jnp.dot(czh_ref[...], sel_bias, preferred_element_type=_F32)
        lm = lax.rem(lax.broadcasted_iota(jnp.int32, (1, lanes), 1), _WIDE)
        scale = jnp.where(lm < HID, 0.5, 1.0).astype(_F32)

        def branch(g_br):
            y = jnp.dot(amat, g_br, preferred_element_type=_F32) + czh_wide
            u = jnp.tanh(y * scale)
            u_rot = jnp.concatenate([u[:, HID:], u[:, :HID]], axis=1)
            f = (0.5 - 0.5 * u) * u_rot
            return jnp.sum(f, axis=0, keepdims=True)

        pool_ad = branch(g_ref[:NUM_NODES, :])
        pool_dis = branch(g_ref[NUM_NODES:, :])

        rsel = (lax.broadcasted_iota(jnp.int32, (B, lanes), 1) // _WIDE
                == lax.broadcasted_iota(jnp.int32, (B, lanes), 0)).astype(_F32)
        gmod = lax.rem(lax.broadcasted_iota(jnp.int32, (lanes, HID), 0), _WIDE)
        gsel = (gmod == lax.broadcasted_iota(
            jnp.int32, (lanes, HID), 1)).astype(_F32)
        pm_ad = jnp.dot(rsel * pool_ad, gsel, preferred_element_type=_F32)
        pm_dis = jnp.dot(rsel * pool_dis, gsel, preferred_element_type=_F32)

        att_v = att_ref[...]
        att_m = jnp.max(att_v, axis=1, keepdims=True)
        att_e = jnp.exp(att_v - att_m)
        p = att_e / jnp.sum(att_e, axis=1, keepdims=True)
        tio = lax.broadcasted_iota(jnp.int32, (B, PERIODS), 1)
        mask = (tio < los_ref[...]).astype(_F32)
        s = jnp.sum(mask * p, axis=1, keepdims=True)

        pooled = (s * pm_ad + (1.0 - s) * pm_dis) * (1.0 / NUM_NODES)
        h1 = jnp.maximum(
            jnp.dot(pooled, c1w_ref[...], preferred_element_type=_F32) + c1b_ref[...],
            0.0,
        )
        out_ref[...] = (
            jnp.dot(h1, c2w_ref[...], preferred_element_type=_F32) + c2b_ref[...]
        )

    return pl.pallas_call(
        body,
        out_shape=jax.ShapeDtypeStruct((B, 2), _F32),
    )(g_wide, amat, czh, att, los, C1W, C1b, C2W, C2b)


def kernel(ad_col_index, dis_col_index, x_batch, LOS_batch, template_edge_index,
           device, emb_tables, Wz, bz, Wr, br, Wh, bh, LzW, Lzb, LrW, Lrb,
           LhW, Lhb, attention, C1W, C1b, C2W, C2b):
    del device, Wr, br, LrW, Lrb

    cols = jnp.concatenate([ad_col_index, dis_col_index]).astype(jnp.int32)
    ef = template_edge_index.astype(_F32)
    dst_row = ef[1].reshape(1, NUM_EDGES)
    src_col = ef[0].reshape(NUM_EDGES, 1)
    dst_col = ef[1].reshape(NUM_EDGES, 1)

    tzh3, amat, czh, xs_t = _tc_prep(
        emb_tables, x_batch.astype(jnp.int32),
        Wz, LzW[:HID], bz.reshape(1, HID), Lzb.reshape(1, HID),
        Wh, LhW[:HID], bh.reshape(1, HID), Lhb.reshape(1, HID),
        dst_row, src_col, dst_col,
    )
    tzh = tzh3.reshape(NUM_COLS * CATS_PAD, _WIDE)

    gathered = _sc_gather(cols, xs_t, tzh)
    g_wide = gathered.reshape(2 * NUM_NODES, B * _WIDE)

    return _tc_main(
        g_wide, amat, czh,
        attention.reshape(1, PERIODS),
        LOS_batch.astype(jnp.int32).reshape(B, 1),
        C1W, C1b.reshape(1, 2 * HID), C2W, C2b.reshape(1, 2),
    )


# speedup vs baseline: 1.1340x; 1.1340x over previous
import functools

import jax
import jax.numpy as jnp
from jax import lax
from jax.experimental import pallas as pl
from jax.experimental.pallas import tpu as pltpu
from jax.experimental.pallas import tpu_sc as plsc

B = 128
NUM_COLS = 256
NUM_NODES = 128
NUM_CATS = 100
CATS_PAD = 128
EMB = 64
HID = 64
PERIODS = 37
NUM_EDGES = 4096

_F32 = jnp.float32
_WIDE = 2 * HID
_CBLK = 8
_NSTEPS = NUM_COLS // _CBLK


def _tc_prep(emb_r, x_batch, Wz, LzWt, bz, Lzb, Wh, LhWt, bh, Lhb,
             dst_row, src_col, dst_col):
    def body(emb_ref, x_ref, wz_ref, lzt_ref, bz_ref, lzb_ref, wh_ref,
             lht_ref, bh_ref, lhb_ref, dstr_ref, srcc_ref, dstc_ref,
             tzh_ref, amat_ref, czh_ref, xst_ref):
        mz = jnp.dot(wz_ref[...], lzt_ref[...], preferred_element_type=_F32)
        mh = jnp.dot(wh_ref[...], lht_ref[...], preferred_element_type=_F32)
        mzh = jnp.concatenate([mz, mh], axis=1)
        tzh_ref[...] = jnp.dot(emb_ref[...], mzh, preferred_element_type=_F32)

        cz = jnp.dot(bz_ref[...], lzt_ref[...],
                     preferred_element_type=_F32) + lzb_ref[...]
        ch = jnp.dot(bh_ref[...], lht_ref[...],
                     preferred_element_type=_F32) + lhb_ref[...]
        czh_ref[...] = jnp.concatenate([cz, ch], axis=1)

        dstr = dstr_ref[...].astype(jnp.int32)
        srcc = srcc_ref[...].astype(jnp.int32)
        dstc = dstc_ref[...].astype(jnp.int32)
        iota_i = lax.broadcasted_iota(jnp.int32, (NUM_NODES, NUM_EDGES), 0)
        odt = (iota_i == dstr).astype(_F32)
        iota_j = lax.broadcasted_iota(jnp.int32, (NUM_EDGES, NUM_NODES), 1)
        os_ = (iota_j == srcc).astype(_F32)
        od2 = (iota_j == dstc).astype(_F32)
        cnt = jnp.dot(odt, os_, preferred_element_type=_F32)
        deg_col = jnp.sum(odt, axis=1, keepdims=True) + 1.0
        deg_row = jnp.sum(od2, axis=0, keepdims=True) + 1.0
        dinv_col = lax.rsqrt(deg_col)
        dinv_row = lax.rsqrt(deg_row)
        ii = lax.broadcasted_iota(jnp.int32, (NUM_NODES, NUM_NODES), 0)
        jj = lax.broadcasted_iota(jnp.int32, (NUM_NODES, NUM_NODES), 1)
        eye = (ii == jj).astype(_F32)
        amat_ref[...] = (cnt + eye) * dinv_col * dinv_row

        xt = jnp.transpose(x_ref[...].astype(_F32)).astype(jnp.int32)
        xst_ref[...] = xt + NUM_CATS * lax.broadcasted_iota(
            jnp.int32, (NUM_COLS, B), 0)

    return pl.pallas_call(
        body,
        out_shape=(
            jax.ShapeDtypeStruct((NUM_COLS * NUM_CATS, _WIDE), _F32),
            jax.ShapeDtypeStruct((NUM_NODES, NUM_NODES), _F32),
            jax.ShapeDtypeStruct((1, _WIDE), _F32),
            jax.ShapeDtypeStruct((NUM_COLS, B), jnp.int32),
        ),
    )(emb_r, x_batch, Wz, LzWt, bz, Lzb, Wh, LhWt, bh, Lhb,
      dst_row, src_col, dst_col)


def _sc_gather(cols_ad_dis, xs_t, tzh):
    info = plsc.get_sparse_core_info()
    nc = info.num_cores
    rows_total = 2 * NUM_NODES
    npw = 8
    half = npw // 2
    nworkers = rows_total // npw

    mesh = plsc.VectorSubcoreMesh(core_axis_name="c", subcore_axis_name="s")

    @functools.partial(
        pl.kernel,
        mesh=mesh,
        compiler_params=pltpu.CompilerParams(use_tc_tiling_on_sc=False),
        out_type=jax.ShapeDtypeStruct((rows_total * B, _WIDE), _F32),
        scratch_types=[
            pltpu.VMEM((npw,), jnp.int32),
            pltpu.VMEM((npw, B), jnp.int32),
            pltpu.VMEM((half * B, _WIDE), _F32),
            pltpu.SemaphoreType.DMA,
        ],
    )
    def body(cols_h, xst_h, tzh_h, out_h, myc_v, idx_v, rows_v, sem):
        wid = lax.axis_index("s") * nc + lax.axis_index("c")

        @pl.when(wid < nworkers)
        def _():
            base = wid * npw
            pltpu.sync_copy(cols_h.at[pl.ds(base, npw)], myc_v)
            pltpu.async_copy(xst_h.at[myc_v], idx_v, sem).wait()
            for p in range(2):
                cps = [
                    pltpu.async_copy(
                        tzh_h.at[idx_v.at[p * half + j]],
                        rows_v.at[pl.ds(j * B, B)],
                        sem,
                    )
                    for j in range(half)
                ]
                for cp in cps:
                    cp.wait()
                pltpu.sync_copy(
                    rows_v,
                    out_h.at[pl.ds((base + p * half) * B, half * B)],
                )

    return body(cols_ad_dis, xs_t, tzh)


def _tc_main(g_wide, amat, czh, att, los, C1W, C1b, C2W, C2b):
    lanes = B * _WIDE

    def body(g_ref, amat_ref, czh_ref, att_ref, los_ref,
             c1w_ref, c1b_ref, c2w_ref, c2b_ref, out_ref):
        amat = amat_ref[...]
        lane_mod = lax.rem(lax.broadcasted_iota(jnp.int32, (_WIDE, lanes), 1),
                           _WIDE)
        sel_bias = (lane_mod == lax.broadcasted_iota(
            jnp.int32, (_WIDE, lanes), 0)).astype(_F32)
        czh_wide = jnp.dot(czh_ref[...], sel_bias, preferred_element_type=_F32)
        lm = lax.rem(lax.broadcasted_iota(jnp.int32, (1, lanes), 1), _WIDE)
        scale = jnp.where(lm < HID, 0.5, 1.0).astype(_F32)

        def branch(g_br):
            y = jnp.dot(amat, g_br, preferred_element_type=_F32) + czh_wide
            u = jnp.tanh(y * scale)
            u_rot = jnp.concatenate([u[:, HID:], u[:, :HID]], axis=1)
            f = (0.5 - 0.5 * u) * u_rot
            return jnp.sum(f, axis=0, keepdims=True)

        pool_ad = branch(g_ref[:NUM_NODES, :])
        pool_dis = branch(g_ref[NUM_NODES:, :])

        rsel = (lax.broadcasted_iota(jnp.int32, (B, lanes), 1) // _WIDE
                == lax.broadcasted_iota(jnp.int32, (B, lanes), 0)).astype(_F32)
        gmod = lax.rem(lax.broadcasted_iota(jnp.int32, (lanes, HID), 0), _WIDE)
        gsel = (gmod == lax.broadcasted_iota(
            jnp.int32, (lanes, HID), 1)).astype(_F32)
        pm_ad = jnp.dot(rsel * pool_ad, gsel, preferred_element_type=_F32)
        pm_dis = jnp.dot(rsel * pool_dis, gsel, preferred_element_type=_F32)

        att_v = att_ref[...]
        att_m = jnp.max(att_v, axis=1, keepdims=True)
        att_e = jnp.exp(att_v - att_m)
        p = att_e / jnp.sum(att_e, axis=1, keepdims=True)
        tio = lax.broadcasted_iota(jnp.int32, (B, PERIODS), 1)
        mask = (tio < los_ref[...]).astype(_F32)
        s = jnp.sum(mask * p, axis=1, keepdims=True)

        pooled = (s * pm_ad + (1.0 - s) * pm_dis) * (1.0 / NUM_NODES)
        h1 = jnp.maximum(
            jnp.dot(pooled, c1w_ref[...], preferred_element_type=_F32) + c1b_ref[...],
            0.0,
        )
        out_ref[...] = (
            jnp.dot(h1, c2w_ref[...], preferred_element_type=_F32) + c2b_ref[...]
        )

    return pl.pallas_call(
        body,
        out_shape=jax.ShapeDtypeStruct((B, 2), _F32),
    )(g_wide, amat, czh, att, los, C1W, C1b, C2W, C2b)


def kernel(ad_col_index, dis_col_index, x_batch, LOS_batch, template_edge_index,
           device, emb_tables, Wz, bz, Wr, br, Wh, bh, LzW, Lzb, LrW, Lrb,
           LhW, Lhb, attention, C1W, C1b, C2W, C2b):
    del device, Wr, br, LrW, Lrb

    cols = jnp.concatenate([ad_col_index, dis_col_index]).astype(jnp.int32)
    ef = template_edge_index.astype(_F32)
    dst_row = ef[1].reshape(1, NUM_EDGES)
    src_col = ef[0].reshape(NUM_EDGES, 1)
    dst_col = ef[1].reshape(NUM_EDGES, 1)

    tzh, amat, czh, xs_t = _tc_prep(
        emb_tables.reshape(NUM_COLS * NUM_CATS, EMB), x_batch.astype(jnp.int32),
        Wz, LzW[:HID], bz.reshape(1, HID), Lzb.reshape(1, HID),
        Wh, LhW[:HID], bh.reshape(1, HID), Lhb.reshape(1, HID),
        dst_row, src_col, dst_col,
    )

    gathered = _sc_gather(cols, xs_t, tzh)
    g_wide = gathered.reshape(2 * NUM_NODES, B * _WIDE)

    return _tc_main(
        g_wide, amat, czh,
        attention.reshape(1, PERIODS),
        LOS_batch.astype(jnp.int32).reshape(B, 1),
        C1W, C1b.reshape(1, 2 * HID), C2W, C2b.reshape(1, 2),
    )


# speedup vs baseline: 1.1362x; 1.0020x over previous
import functools

import jax
import jax.numpy as jnp
from jax import lax
from jax.experimental import pallas as pl
from jax.experimental.pallas import tpu as pltpu
from jax.experimental.pallas import tpu_sc as plsc

B = 128
NUM_COLS = 256
NUM_NODES = 128
NUM_CATS = 100
CATS_PAD = 128
EMB = 64
HID = 64
PERIODS = 37
NUM_EDGES = 4096

_F32 = jnp.float32
_WIDE = 2 * HID
_CBLK = 8
_NSTEPS = NUM_COLS // _CBLK


def _tc_prep(emb_r, x_batch, Wz, LzWt, bz, Lzb, Wh, LhWt, bh, Lhb,
             dst_row, src_col, dst_col):
    def body(emb_ref, x_ref, wz_ref, lzt_ref, bz_ref, lzb_ref, wh_ref,
             lht_ref, bh_ref, lhb_ref, dstr_ref, srcc_ref, dstc_ref,
             tzh_ref, amat_ref, czh_ref, xst_ref):
        mz = jnp.dot(wz_ref[...], lzt_ref[...], preferred_element_type=_F32)
        mh = jnp.dot(wh_ref[...], lht_ref[...], preferred_element_type=_F32)
        mzh = jnp.concatenate([mz, mh], axis=1)
        tzh_ref[...] = jnp.reshape(
            jnp.dot(emb_ref[...], mzh, preferred_element_type=_F32),
            (NUM_COLS * NUM_CATS // B, B, _WIDE))

        cz = jnp.dot(bz_ref[...], lzt_ref[...],
                     preferred_element_type=_F32) + lzb_ref[...]
        ch = jnp.dot(bh_ref[...], lht_ref[...],
                     preferred_element_type=_F32) + lhb_ref[...]
        czh_ref[...] = jnp.concatenate([cz, ch], axis=1)

        dstr = dstr_ref[...].astype(jnp.int32)
        srcc = srcc_ref[...].astype(jnp.int32)
        dstc = dstc_ref[...].astype(jnp.int32)
        iota_i = lax.broadcasted_iota(jnp.int32, (NUM_NODES, NUM_EDGES), 0)
        odt = (iota_i == dstr).astype(_F32)
        iota_j = lax.broadcasted_iota(jnp.int32, (NUM_EDGES, NUM_NODES), 1)
        os_ = (iota_j == srcc).astype(_F32)
        od2 = (iota_j == dstc).astype(_F32)
        cnt = jnp.dot(odt, os_, preferred_element_type=_F32)
        deg_col = jnp.sum(odt, axis=1, keepdims=True) + 1.0
        deg_row = jnp.sum(od2, axis=0, keepdims=True) + 1.0
        dinv_col = lax.rsqrt(deg_col)
        dinv_row = lax.rsqrt(deg_row)
        ii = lax.broadcasted_iota(jnp.int32, (NUM_NODES, NUM_NODES), 0)
        jj = lax.broadcasted_iota(jnp.int32, (NUM_NODES, NUM_NODES), 1)
        eye = (ii == jj).astype(_F32)
        amat_ref[...] = (cnt + eye) * dinv_col * dinv_row

        xt = jnp.transpose(x_ref[...].astype(_F32)).astype(jnp.int32)
        xst_ref[...] = xt + NUM_CATS * lax.broadcasted_iota(
            jnp.int32, (NUM_COLS, B), 0)

    return pl.pallas_call(
        body,
        out_shape=(
            jax.ShapeDtypeStruct((NUM_COLS * NUM_CATS // B, B, _WIDE), _F32),
            jax.ShapeDtypeStruct((NUM_NODES, NUM_NODES), _F32),
            jax.ShapeDtypeStruct((1, _WIDE), _F32),
            jax.ShapeDtypeStruct((NUM_COLS, B), jnp.int32),
        ),
    )(emb_r, x_batch, Wz, LzWt, bz, Lzb, Wh, LhWt, bh, Lhb,
      dst_row, src_col, dst_col)


def _sc_gather(cols_ad_dis, xs_t, tzh):
    info = plsc.get_sparse_core_info()
    nc = info.num_cores
    rows_total = 2 * NUM_NODES
    npw = 8
    half = npw // 2
    nworkers = rows_total // npw

    mesh = plsc.VectorSubcoreMesh(core_axis_name="c", subcore_axis_name="s")

    @functools.partial(
        pl.kernel,
        mesh=mesh,
        compiler_params=pltpu.CompilerParams(use_tc_tiling_on_sc=False),
        out_type=jax.ShapeDtypeStruct((rows_total * B, _WIDE), _F32),
        scratch_types=[
            pltpu.VMEM((npw,), jnp.int32),
            pltpu.VMEM((npw, B), jnp.int32),
            pltpu.VMEM((half * B, _WIDE), _F32),
            pltpu.SemaphoreType.DMA,
        ],
    )
    def body(cols_h, xst_h, tzh_h, out_h, myc_v, idx_v, rows_v, sem):
        wid = lax.axis_index("s") * nc + lax.axis_index("c")

        @pl.when(wid < nworkers)
        def _():
            base = wid * npw
            pltpu.sync_copy(cols_h.at[pl.ds(base, npw)], myc_v)
            pltpu.async_copy(xst_h.at[myc_v], idx_v, sem).wait()
            for p in range(2):
                cps = [
                    pltpu.async_copy(
                        tzh_h.at[idx_v.at[p * half + j]],
                        rows_v.at[pl.ds(j * B, B)],
                        sem,
                    )
                    for j in range(half)
                ]
                for cp in cps:
                    cp.wait()
                pltpu.sync_copy(
                    rows_v,
                    out_h.at[pl.ds((base + p * half) * B, half * B)],
                )

    return body(cols_ad_dis, xs_t, tzh)


def _tc_main(g_wide, amat, czh, att, los, C1W, C1b, C2W, C2b):
    lanes = B * _WIDE

    def body(g_ref, amat_ref, czh_ref, att_ref, los_ref,
             c1w_ref, c1b_ref, c2w_ref, c2b_ref, out_ref):
        amat = amat_ref[...]
        lane_mod = lax.rem(lax.broadcasted_iota(jnp.int32, (_WIDE, lanes), 1),
                           _WIDE)
        sel_bias = (lane_mod == lax.broadcasted_iota(
            jnp.int32, (_WIDE, lanes), 0)).astype(_F32)
        czh_wide = jnp.dot(czh_ref[...], sel_bias, preferred_element_type=_F32)
        lm = lax.rem(lax.broadcasted_iota(jnp.int32, (1, lanes), 1), _WIDE)
        scale = jnp.where(lm < HID, 0.5, 1.0).astype(_F32)

        def branch(g_br):
            y = jnp.dot(amat, g_br, preferred_element_type=_F32) + czh_wide
            u = jnp.tanh(y * scale)
            u_rot = jnp.concatenate([u[:, HID:], u[:, :HID]], axis=1)
            f = (0.5 - 0.5 * u) * u_rot
            return jnp.sum(f, axis=0, keepdims=True)

        pool_ad = branch(g_ref[:NUM_NODES, :])
        pool_dis = branch(g_ref[NUM_NODES:, :])

        rsel = (lax.broadcasted_iota(jnp.int32, (B, lanes), 1) // _WIDE
                == lax.broadcasted_iota(jnp.int32, (B, lanes), 0)).astype(_F32)
        gmod = lax.rem(lax.broadcasted_iota(jnp.int32, (lanes, HID), 0), _WIDE)
        gsel = (gmod == lax.broadcasted_iota(
            jnp.int32, (lanes, HID), 1)).astype(_F32)
        pm_ad = jnp.dot(rsel * pool_ad, gsel, preferred_element_type=_F32)
        pm_dis = jnp.dot(rsel * pool_dis, gsel, preferred_element_type=_F32)

        att_v = att_ref[...]
        att_m = jnp.max(att_v, axis=1, keepdims=True)
        att_e = jnp.exp(att_v - att_m)
        p = att_e / jnp.sum(att_e, axis=1, keepdims=True)
        tio = lax.broadcasted_iota(jnp.int32, (B, PERIODS), 1)
        mask = (tio < los_ref[...]).astype(_F32)
        s = jnp.sum(mask * p, axis=1, keepdims=True)

        pooled = (s * pm_ad + (1.0 - s) * pm_dis) * (1.0 / NUM_NODES)
        h1 = jnp.maximum(
            jnp.dot(pooled, c1w_ref[...], preferred_element_type=_F32) + c1b_ref[...],
            0.0,
        )
        out_ref[...] = (
            jnp.dot(h1, c2w_ref[...], preferred_element_type=_F32) + c2b_ref[...]
        )

    return pl.pallas_call(
        body,
        out_shape=jax.ShapeDtypeStruct((B, 2), _F32),
    )(g_wide, amat, czh, att, los, C1W, C1b, C2W, C2b)


def kernel(ad_col_index, dis_col_index, x_batch, LOS_batch, template_edge_index,
           device, emb_tables, Wz, bz, Wr, br, Wh, bh, LzW, Lzb, LrW, Lrb,
           LhW, Lhb, attention, C1W, C1b, C2W, C2b):
    del device, Wr, br, LrW, Lrb

    cols = jnp.concatenate([ad_col_index, dis_col_index]).astype(jnp.int32)
    ef = template_edge_index.astype(_F32)
    dst_row = ef[1].reshape(1, NUM_EDGES)
    src_col = ef[0].reshape(NUM_EDGES, 1)
    dst_col = ef[1].reshape(NUM_EDGES, 1)

    tzh3, amat, czh, xs_t = _tc_prep(
        emb_tables.reshape(NUM_COLS * NUM_CATS, EMB), x_batch.astype(jnp.int32),
        Wz, LzW[:HID], bz.reshape(1, HID), Lzb.reshape(1, HID),
        Wh, LhW[:HID], bh.reshape(1, HID), Lhb.reshape(1, HID),
        dst_row, src_col, dst_col,
    )
    tzh = tzh3.reshape(NUM_COLS * NUM_CATS, _WIDE)

    gathered = _sc_gather(cols, xs_t, tzh)
    g_wide = gathered.reshape(2 * NUM_NODES, B * _WIDE)

    return _tc_main(
        g_wide, amat, czh,
        attention.reshape(1, PERIODS),
        LOS_batch.astype(jnp.int32).reshape(B, 1),
        C1W, C1b.reshape(1, 2 * HID), C2W, C2b.reshape(1, 2),
    )
